# baseline (speedup 1.0000x reference)
import functools

import jax
import jax.numpy as jnp
from jax import lax
from jax.experimental import pallas as pl
from jax.experimental.pallas import tpu as pltpu
from jax.experimental.pallas import tpu_sc as plsc

NC, NS = 2, 16
NW = NC * NS
CI = 128
NB = 2


def kernel(x, table):
    b, s = x.shape
    vocab, d = table.shape
    n = b * s
    scale = float(d) ** 0.5
    nrows = n // CI
    nch = nrows // NW

    x2 = x.reshape(NW, nch, CI)
    p = jnp.arange(n, dtype=jnp.int32)
    omap = ((p % s) * b + p // s).reshape(NW, nch, CI)

    mesh = plsc.VectorSubcoreMesh(core_axis_name="c", subcore_axis_name="s")

    @functools.partial(
        pl.kernel, mesh=mesh,
        out_type=jax.ShapeDtypeStruct((n, d), table.dtype),
        scratch_types=(
            [pltpu.VMEM((nch, CI), jnp.int32) for _ in range(2)]
            + [pltpu.VMEM((CI, d), jnp.float32) for _ in range(2 * NB)]
            + [pltpu.SemaphoreType.DMA for _ in range(2 * NB + 1)]
        ),
    )
    def emb_kernel(t_hbm, xf_hbm, om_hbm, o_hbm, idx_v, om_v,
                   *bufs_and_sems):
        gbuf = bufs_and_sems[:NB]
        obuf = bufs_and_sems[NB:2 * NB]
        gsem = bufs_and_sems[2 * NB:3 * NB]
        osem = bufs_and_sems[3 * NB:4 * NB]
        isem = bufs_and_sems[4 * NB]

        wid = lax.axis_index("s") * NC + lax.axis_index("c")

        pltpu.async_copy(xf_hbm.at[wid], idx_v, isem)
        pltpu.async_copy(om_hbm.at[wid], om_v, isem)
        pltpu.make_async_copy(xf_hbm.at[wid], idx_v, isem).wait()
        pltpu.make_async_copy(om_hbm.at[wid], om_v, isem).wait()

        def issue_gather(j, c):
            pltpu.async_copy(t_hbm.at[idx_v.at[c]], gbuf[j], gsem[j])

        def wait_gather(j, c):
            pltpu.make_async_copy(t_hbm.at[idx_v.at[c]], gbuf[j],
                                  gsem[j]).wait()

        def issue_out(j, c):
            pltpu.async_copy(obuf[j], o_hbm.at[om_v.at[c]], osem[j])

        def wait_out(j, c):
            pltpu.make_async_copy(obuf[j], o_hbm.at[om_v.at[c]],
                                  osem[j]).wait()

        for j in range(NB):
            issue_gather(j, j)

        @pl.loop(0, nch, step=NB)
        def _(c0):
            for j in range(NB):
                c = c0 + j
                wait_gather(j, c)

                @pl.when(c >= NB)
                def _():
                    wait_out(j, c - NB)

                @plsc.parallel_loop(0, CI, unroll=2)
                def _(r):
                    for cc in range(0, d, 16):
                        obuf[j][r, pl.ds(cc, 16)] = (
                            gbuf[j][r, pl.ds(cc, 16)] * scale)

                @pl.when(c + NB < nch)
                def _():
                    issue_gather(j, c + NB)

                issue_out(j, c)

        for j in range(NB):
            wait_out(j, nch - NB + j)

    y = emb_kernel(table, x2, omap)
    return y.reshape(s, b, d).transpose(1, 0, 2)


# speedup vs baseline: 1.0031x; 1.0031x over previous
import functools

import jax
import jax.numpy as jnp
import numpy as np
from jax import lax
from jax.experimental import pallas as pl
from jax.experimental.pallas import tpu as pltpu
from jax.experimental.pallas import tpu_sc as plsc

NC, NS = 2, 16
NW = NC * NS
CI = 128
NB = 2


def kernel(x, table):
    b, s = x.shape
    vocab, d = table.shape
    n = b * s
    scale = float(d) ** 0.5
    nrows = n // CI
    nch = nrows // NW

    x2 = x.reshape(NW, nch, CI)
    p = np.arange(n, dtype=np.int32)
    omap = jnp.asarray(((p % s) * b + p // s).reshape(NW, nch, CI))

    mesh = plsc.VectorSubcoreMesh(core_axis_name="c", subcore_axis_name="s")

    @functools.partial(
        pl.kernel, mesh=mesh,
        out_type=jax.ShapeDtypeStruct((n, d), table.dtype),
        scratch_types=(
            [pltpu.VMEM((nch, CI), jnp.int32) for _ in range(2)]
            + [pltpu.VMEM((CI, d), jnp.float32) for _ in range(2 * NB)]
            + [pltpu.SemaphoreType.DMA for _ in range(2 * NB + 1)]
        ),
    )
    def emb_kernel(t_hbm, xf_hbm, om_hbm, o_hbm, idx_v, om_v,
                   *bufs_and_sems):
        gbuf = bufs_and_sems[:NB]
        obuf = bufs_and_sems[NB:2 * NB]
        gsem = bufs_and_sems[2 * NB:3 * NB]
        osem = bufs_and_sems[3 * NB:4 * NB]
        isem = bufs_and_sems[4 * NB]

        wid = lax.axis_index("s") * NC + lax.axis_index("c")

        pltpu.async_copy(xf_hbm.at[wid], idx_v, isem)
        pltpu.async_copy(om_hbm.at[wid], om_v, isem)
        pltpu.make_async_copy(xf_hbm.at[wid], idx_v, isem).wait()
        pltpu.make_async_copy(om_hbm.at[wid], om_v, isem).wait()

        def issue_gather(j, c):
            pltpu.async_copy(t_hbm.at[idx_v.at[c]], gbuf[j], gsem[j])

        def wait_gather(j, c):
            pltpu.make_async_copy(t_hbm.at[idx_v.at[c]], gbuf[j],
                                  gsem[j]).wait()

        def issue_out(j, c):
            pltpu.async_copy(obuf[j], o_hbm.at[om_v.at[c]], osem[j])

        def wait_out(j, c):
            pltpu.make_async_copy(obuf[j], o_hbm.at[om_v.at[c]],
                                  osem[j]).wait()

        for j in range(NB):
            issue_gather(j, j)

        @pl.loop(0, nch, step=NB)
        def _(c0):
            for j in range(NB):
                c = c0 + j
                wait_gather(j, c)

                @pl.when(c >= NB)
                def _():
                    wait_out(j, c - NB)

                @pl.loop(0, CI)
                def _(r):
                    for cc in range(0, d, 16):
                        obuf[j][r, pl.ds(cc, 16)] = (
                            gbuf[j][r, pl.ds(cc, 16)] * scale)

                @pl.when(c + NB < nch)
                def _():
                    issue_gather(j, c + NB)

                issue_out(j, c)

        for j in range(NB):
            wait_out(j, nch - NB + j)

    y = emb_kernel(table, x2, omap)
    return y.reshape(s, b, d).transpose(1, 0, 2)


# speedup vs baseline: 1.0162x; 1.0130x over previous
import functools

import jax
import jax.numpy as jnp
import numpy as np
from jax import lax
from jax.experimental import pallas as pl
from jax.experimental.pallas import tpu as pltpu
from jax.experimental.pallas import tpu_sc as plsc

NC, NS = 2, 16
NW = NC * NS
CI = 64
NB = 4


def kernel(x, table):
    b, s = x.shape
    vocab, d = table.shape
    n = b * s
    scale = float(d) ** 0.5
    nrows = n // CI
    nch = nrows // NW

    x2 = x.reshape(NW, nch, CI)
    p = np.arange(n, dtype=np.int32)
    omap = jnp.asarray(((p % s) * b + p // s).reshape(NW, nch, CI))

    mesh = plsc.VectorSubcoreMesh(core_axis_name="c", subcore_axis_name="s")

    @functools.partial(
        pl.kernel, mesh=mesh,
        out_type=jax.ShapeDtypeStruct((n, d), table.dtype),
        scratch_types=(
            [pltpu.VMEM((nch, CI), jnp.int32) for _ in range(2)]
            + [pltpu.VMEM((CI, d), jnp.float32) for _ in range(2 * NB)]
            + [pltpu.SemaphoreType.DMA for _ in range(2 * NB + 1)]
        ),
    )
    def emb_kernel(t_hbm, xf_hbm, om_hbm, o_hbm, idx_v, om_v,
                   *bufs_and_sems):
        gbuf = bufs_and_sems[:NB]
        obuf = bufs_and_sems[NB:2 * NB]
        gsem = bufs_and_sems[2 * NB:3 * NB]
        osem = bufs_and_sems[3 * NB:4 * NB]
        isem = bufs_and_sems[4 * NB]

        wid = lax.axis_index("s") * NC + lax.axis_index("c")

        pltpu.async_copy(xf_hbm.at[wid], idx_v, isem)
        pltpu.async_copy(om_hbm.at[wid], om_v, isem)
        pltpu.make_async_copy(xf_hbm.at[wid], idx_v, isem).wait()
        pltpu.make_async_copy(om_hbm.at[wid], om_v, isem).wait()

        def issue_gather(j, c):
            pltpu.async_copy(t_hbm.at[idx_v.at[c]], gbuf[j], gsem[j])

        def wait_gather(j, c):
            pltpu.make_async_copy(t_hbm.at[idx_v.at[c]], gbuf[j],
                                  gsem[j]).wait()

        def issue_out(j, c):
            pltpu.async_copy(obuf[j], o_hbm.at[om_v.at[c]], osem[j])

        def wait_out(j, c):
            pltpu.make_async_copy(obuf[j], o_hbm.at[om_v.at[c]],
                                  osem[j]).wait()

        for j in range(NB):
            issue_gather(j, j)

        @pl.loop(0, nch, step=NB)
        def _(c0):
            for j in range(NB):
                c = c0 + j
                wait_gather(j, c)

                @pl.when(c >= NB)
                def _():
                    wait_out(j, c - NB)

                @pl.loop(0, CI)
                def _(r):
                    for cc in range(0, d, 16):
                        obuf[j][r, pl.ds(cc, 16)] = (
                            gbuf[j][r, pl.ds(cc, 16)] * scale)

                @pl.when(c + NB < nch)
                def _():
                    issue_gather(j, c + NB)

                issue_out(j, c)

        for j in range(NB):
            wait_out(j, nch - NB + j)

    y = emb_kernel(table, x2, omap)
    return y.reshape(s, b, d).transpose(1, 0, 2)
